# baseline (speedup 1.0000x reference)
import functools

import jax
import jax.numpy as jnp
from jax import lax
from jax.experimental import pallas as pl
from jax.experimental.pallas import tpu as pltpu
from jax.experimental.pallas import tpu_sc as plsc

_NC = 2
_NS = 16
_NW = _NC * _NS
_NQ = 4
_CH = 32
_RB = 2560


def _sc_mesh():
    return plsc.VectorSubcoreMesh(core_axis_name="c", subcore_axis_name="s")


def _sc_degree(dst2, zeros16, ones16, Kd):
    _, CH = dst2.shape
    PADN = zeros16.shape[0]

    @functools.partial(
        pl.kernel,
        mesh=_sc_mesh(),
        out_type=jax.ShapeDtypeStruct((_NC, PADN, 16), jnp.float32),
        scratch_types=[
            pltpu.VMEM((CH,), jnp.int32),
            pltpu.VMEM((CH, 16), jnp.float32),
            pltpu.VMEM_SHARED((PADN, 16), jnp.float32),
        ],
    )
    def deg_kernel(dst_hbm, zeros_hbm, ones_hbm, out_hbm, dstv, onesv, acc):
        c = lax.axis_index("c")
        s = lax.axis_index("s")
        wid = s * _NC + c
        pltpu.sync_copy(ones_hbm, onesv)

        @pl.when(s == 0)
        def _init():
            pltpu.sync_copy(zeros_hbm, acc)

        plsc.subcore_barrier()

        def step(g, carry):
            pltpu.sync_copy(dst_hbm.at[wid * Kd + g], dstv)
            pltpu.sync_copy(onesv, acc.at[dstv], add=True)
            return carry

        lax.fori_loop(0, Kd, step, 0)
        plsc.subcore_barrier()

        @pl.when(s == 0)
        def _out():
            pltpu.sync_copy(acc, out_hbm.at[c])

    return deg_kernel(dst2, zeros16, ones16)


def _sc_scatter(hs, src4, dst4, nchunk):
    PADN, D = hs.shape
    H4 = PADN // _NQ
    CH = src4.shape[-1]

    @functools.partial(
        pl.kernel,
        mesh=_sc_mesh(),
        out_type=jax.ShapeDtypeStruct((_NQ, H4, D), jnp.float32),
        scratch_types=[
            pltpu.VMEM((CH,), jnp.int32),
            pltpu.VMEM((CH,), jnp.int32),
            pltpu.VMEM((CH, D), jnp.float32),
            pltpu.VMEM_SHARED((H4, D), jnp.float32),
        ],
    )
    def scat_kernel(hs_hbm, src_hbm, dst_hbm, out_hbm, srcv, dstv, buf, acc):
        c = lax.axis_index("c")
        s = lax.axis_index("s")

        for q in range(2):
            wid = (c * 2 + q) * _NS + s

            @pl.when(s == 0)
            def _init():
                pltpu.sync_copy(hs_hbm.at[pl.ds((c * 2 + q) * H4, H4)], acc)

            plsc.subcore_barrier()

            def step(g, carry):
                pltpu.sync_copy(src_hbm.at[wid * nchunk + g], srcv)
                pltpu.sync_copy(dst_hbm.at[wid * nchunk + g], dstv)
                pltpu.sync_copy(hs_hbm.at[srcv], buf)
                pltpu.sync_copy(buf, acc.at[dstv], add=True)
                return carry

            lax.fori_loop(0, nchunk, step, 0)
            plsc.subcore_barrier()

            @pl.when(s == 0)
            def _out():
                pltpu.sync_copy(acc, out_hbm.at[c * 2 + q])

            plsc.subcore_barrier()

    return scat_kernel(hs, src4, dst4)


def _dsp_block(deg_ref):
    deg = deg_ref[0] + deg_ref[1]
    return lax.rsqrt(deg[:, :1] + 1.0)


def _tc_first(x_pad, W1, degp):
    PADN, D = x_pad.shape
    R = _RB

    def body(x_ref, w_ref, deg_ref, hs_ref):
        h = jnp.dot(x_ref[...], w_ref[...], preferred_element_type=jnp.float32)
        hs_ref[...] = h * _dsp_block(deg_ref)

    return pl.pallas_call(
        body,
        grid=(PADN // R,),
        in_specs=[
            pl.BlockSpec((R, D), lambda i: (i, 0)),
            pl.BlockSpec((D, D), lambda i: (0, 0)),
            pl.BlockSpec((2, R, 16), lambda i: (0, i, 0)),
        ],
        out_specs=pl.BlockSpec((R, D), lambda i: (i, 0)),
        out_shape=jax.ShapeDtypeStruct((PADN, D), jnp.float32),
    )(x_pad, W1, degp)


def _tc_mid(p, degp, b1r, W2, n_real):
    PADN, D = p.shape
    R = _RB

    def body(p_ref, deg_ref, b_ref, w_ref, o_ref):
        i = pl.program_id(0)
        dsp = _dsp_block(deg_ref)
        z = jnp.maximum(p_ref[...] * dsp + b_ref[...], 0.0)
        h2 = jnp.dot(z, w_ref[...], preferred_element_type=jnp.float32)
        hs2 = h2 * dsp
        rows = i * R + lax.broadcasted_iota(jnp.int32, (R, 1), 0)
        o_ref[...] = jnp.where(rows < n_real, hs2, 0.0)

    return pl.pallas_call(
        body,
        grid=(PADN // R,),
        in_specs=[
            pl.BlockSpec((R, D), lambda i: (i, 0)),
            pl.BlockSpec((2, R, 16), lambda i: (0, i, 0)),
            pl.BlockSpec((1, D), lambda i: (0, 0)),
            pl.BlockSpec((D, D), lambda i: (0, 0)),
        ],
        out_specs=pl.BlockSpec((R, D), lambda i: (i, 0)),
        out_shape=jax.ShapeDtypeStruct((PADN, D), jnp.float32),
    )(p, degp, b1r, W2)


def _tc_last(p, degp, b2r):
    PADN, D = p.shape
    R = _RB

    def body(p_ref, deg_ref, b_ref, out_ref):
        dsp = _dsp_block(deg_ref)
        out_ref[...] = p_ref[...] * dsp + b_ref[...]

    return pl.pallas_call(
        body,
        grid=(PADN // R,),
        in_specs=[
            pl.BlockSpec((R, D), lambda i: (i, 0)),
            pl.BlockSpec((2, R, 16), lambda i: (0, i, 0)),
            pl.BlockSpec((1, D), lambda i: (0, 0)),
        ],
        out_specs=pl.BlockSpec((R, D), lambda i: (i, 0)),
        out_shape=jax.ShapeDtypeStruct((PADN, D), jnp.float32),
    )(p, degp, b2r)


def kernel(x, edge_index, W1, b1, W2, b2):
    N, D = x.shape
    E = edge_index.shape[1]
    PADN = -(-(N + 1) // (_NQ * _RB)) * (_NQ * _RB)
    H4 = PADN // _NQ

    nchunk = -(-E // (_NS * _CH))
    Lr = _NS * nchunk * _CH
    src = edge_index[0].astype(jnp.int32)
    dst = edge_index[1].astype(jnp.int32)
    padn = jnp.full((Lr - E,), N, jnp.int32)
    padz = jnp.zeros((Lr - E,), jnp.int32)
    srcqs = []
    dstqs = []
    for Q in range(_NQ):
        inq = (dst >= Q * H4) & (dst < (Q + 1) * H4)
        srcqs.append(jnp.concatenate([jnp.where(inq, src, N), padn]))
        dstqs.append(jnp.concatenate([jnp.where(inq, dst - Q * H4, 0), padz]))
    src4 = jnp.stack(srcqs).reshape(_NQ * _NS * nchunk, _CH)
    dst4 = jnp.stack(dstqs).reshape(_NQ * _NS * nchunk, _CH)
    Kd = -(-E // (_NW * _CH))
    EPADd = _NW * Kd * _CH
    dst_pad = jnp.concatenate([dst, jnp.full((EPADd - E,), N, jnp.int32)])
    dst2d = dst_pad.reshape(_NW * Kd, _CH)

    x_pad = jnp.zeros((PADN, D), jnp.float32).at[:N].set(x)
    zeros16 = jnp.zeros((PADN, 16), jnp.float32)
    ones16 = jnp.ones((_CH, 16), jnp.float32)
    b1r = b1.reshape(1, D)
    b2r = b2.reshape(1, D)

    degp = _sc_degree(dst2d, zeros16, ones16, Kd)
    hs1 = _tc_first(x_pad, W1, degp)
    p1 = _sc_scatter(hs1, src4, dst4, nchunk).reshape(PADN, D)
    hs2 = _tc_mid(p1, degp, b1r, W2, N)
    p2 = _sc_scatter(hs2, src4, dst4, nchunk).reshape(PADN, D)
    out_pad = _tc_last(p2, degp, b2r)
    return out_pad[:N]
